# baseline (speedup 1.0000x reference)
import functools

import jax
import jax.numpy as jnp
from jax import lax
from jax.experimental import pallas as pl
from jax.experimental.pallas import tpu as pltpu
from jax.experimental.pallas import tpu_sc as plsc

_MAX_SEQ = 4096
_TOTAL = 32768
_BATCH = 16

_NC = 2
_NS = 16
_NW = _NC * _NS
_CHUNK = (_BATCH * _MAX_SEQ) // _NW
_BUF = _CHUNK + 16
_L = 16


def _pad_body(flat_hbm, cu_hbm, out_hbm, mask_hbm, cu_v, buf_v, val_v, msk_v):
    wid = lax.axis_index("s") * _NC + lax.axis_index("c")
    row = wid // 2
    half = wid % 2
    col0 = half * _CHUNK

    pltpu.sync_copy(cu_hbm.at[pl.ds(0, _L)], cu_v.at[pl.ds(0, _L)])
    cu_pair = cu_v[pl.ds(row, _L)]
    s = cu_pair[0]
    e = jnp.where(row == _BATCH - 1, _TOTAL, cu_pair[1])
    length = jnp.minimum(e - s, _MAX_SEQ)

    start = s + col0
    start_al = jnp.minimum((start // 8) * 8, _TOTAL - _BUF)
    start_al = pl.multiple_of(start_al, 8)
    off = start - start_al

    pltpu.sync_copy(flat_hbm.at[pl.ds(start_al, _BUF)], buf_v)

    lane = lax.iota(jnp.int32, _L)
    idx0 = off + lane
    rel0 = lane + (col0 - length)

    @plsc.parallel_loop(0, _CHUNK, _L, unroll=16)
    def _chunk(base):
        rel = rel0 + base
        valid = rel < 0
        vals = plsc.load_gather(buf_v, [idx0 + base], mask=valid)
        val_v[pl.ds(base, _L)] = jnp.where(valid, vals, 0.0)
        msk_v[pl.ds(base, _L)] = lax.shift_right_logical(rel, 31)

    pltpu.sync_copy(val_v, out_hbm.at[row, pl.ds(col0, _CHUNK)])
    pltpu.sync_copy(msk_v, mask_hbm.at[row, pl.ds(col0, _CHUNK)])


_pad_sc = functools.partial(
    pl.kernel,
    out_type=(
        jax.ShapeDtypeStruct((_BATCH, _MAX_SEQ), jnp.float32),
        jax.ShapeDtypeStruct((_BATCH, _MAX_SEQ), jnp.int32),
    ),
    mesh=plsc.VectorSubcoreMesh(core_axis_name="c", subcore_axis_name="s"),
    compiler_params=pltpu.CompilerParams(needs_layout_passes=False),
    scratch_types=[
        pltpu.VMEM((32,), jnp.int32),
        pltpu.VMEM((_BUF,), jnp.float32),
        pltpu.VMEM((_CHUNK,), jnp.float32),
        pltpu.VMEM((_CHUNK,), jnp.int32),
    ],
)(_pad_body)


def kernel(flat, cu_seqlens):
    return _pad_sc(flat, cu_seqlens)


# speedup vs baseline: 1.0164x; 1.0164x over previous
import functools

import jax
import jax.numpy as jnp
from jax import lax
from jax.experimental import pallas as pl
from jax.experimental.pallas import tpu as pltpu
from jax.experimental.pallas import tpu_sc as plsc

_MAX_SEQ = 4096
_TOTAL = 32768
_BATCH = 16

_NC = 2
_NS = 16
_NW = _NC * _NS
_CHUNK = (_BATCH * _MAX_SEQ) // _NW
_BUF = _CHUNK + 16
_L = 16


def _pad_body(flat_hbm, cu_hbm, out_hbm, cu_v, buf_v, val_v):
    wid = lax.axis_index("s") * _NC + lax.axis_index("c")
    row = wid // 2
    half = wid % 2
    col0 = half * _CHUNK

    pltpu.sync_copy(cu_hbm.at[pl.ds(0, _L)], cu_v.at[pl.ds(0, _L)])
    cu_pair = cu_v[pl.ds(row, _L)]
    s = cu_pair[0]
    e = jnp.where(row == _BATCH - 1, _TOTAL, cu_pair[1])
    length = jnp.minimum(e - s, _MAX_SEQ)

    start = s + col0
    start_al = jnp.minimum((start // 8) * 8, _TOTAL - _BUF)
    start_al = pl.multiple_of(start_al, 8)
    off = start - start_al

    pltpu.sync_copy(flat_hbm.at[pl.ds(start_al, _BUF)], buf_v)

    lane = lax.iota(jnp.int32, _L)
    idx0 = off + lane
    rel0 = lane + (col0 - length)

    @plsc.parallel_loop(0, _CHUNK, _L, unroll=8)
    def _chunk(base):
        valid = (rel0 + base) < 0
        vals = plsc.load_gather(buf_v, [idx0 + base], mask=valid)
        val_v[pl.ds(base, _L)] = jnp.where(valid, vals, 0.0)

    pltpu.sync_copy(val_v, out_hbm.at[row, pl.ds(col0, _CHUNK)])


_pad_sc = functools.partial(
    pl.kernel,
    out_type=jax.ShapeDtypeStruct((_BATCH, _MAX_SEQ), jnp.float32),
    mesh=plsc.VectorSubcoreMesh(core_axis_name="c", subcore_axis_name="s"),
    compiler_params=pltpu.CompilerParams(needs_layout_passes=False),
    scratch_types=[
        pltpu.VMEM((32,), jnp.int32),
        pltpu.VMEM((_BUF,), jnp.float32),
        pltpu.VMEM((_CHUNK,), jnp.float32),
    ],
)(_pad_body)


def _mask_body(lens_ref, mask_ref):
    pos = lax.broadcasted_iota(jnp.int32, (_BATCH, _MAX_SEQ), 1)
    mask_ref[...] = (pos < lens_ref[...]).astype(jnp.int32)


_mask_tc = pl.pallas_call(
    _mask_body,
    out_shape=jax.ShapeDtypeStruct((_BATCH, _MAX_SEQ), jnp.int32),
    in_specs=[pl.BlockSpec(memory_space=pltpu.VMEM)],
    out_specs=pl.BlockSpec(memory_space=pltpu.VMEM),
)


def kernel(flat, cu_seqlens):
    lens = jnp.minimum(
        cu_seqlens[1:] - cu_seqlens[:-1], _MAX_SEQ
    ).astype(jnp.int32)[:, None]
    padded = _pad_sc(flat, cu_seqlens)
    mask = _mask_tc(lens)
    return padded, mask
